# Initial kernel scaffold; baseline (speedup 1.0000x reference)
import jax
import jax.numpy as jnp
from jax.experimental import pallas as pl


def kernel(input, positive, negative, input_logits, negative_logits):
    raise NotImplementedError("write your pallas kernel here")


# speedup vs baseline: 58.5198x; 58.5198x over previous
import jax
import jax.numpy as jnp
from jax.experimental import pallas as pl

TAU = 0.07
K = 50
KPAD = 64


def _iota_f32(shape, dim):
    return jax.lax.broadcasted_iota(jnp.int32, shape, dim).astype(jnp.float32)


def _loss_kernel(xin_ref, xpos_ref, xneg_ref, il_ref, nl_ref, out_ref):
    C, N = xin_ref.shape
    NC = il_ref.shape[0]
    R = NC * KPAD

    xin = xin_ref[...]
    xpos = xpos_ref[...]

    rn_in_raw = jnp.sqrt(jnp.sum(xin * xin, axis=0, keepdims=True))
    n_in = xin / jnp.maximum(rn_in_raw, 1e-12)
    rn_pos_raw = jnp.sqrt(jnp.sum(xpos * xpos, axis=0, keepdims=True))
    n_pos = xpos / jnp.maximum(rn_pos_raw, 1e-12)
    rni = jnp.sqrt(jnp.sum(n_in * n_in, axis=0, keepdims=True))
    rnp = jnp.sqrt(jnp.sum(n_pos * n_pos, axis=0, keepdims=True))

    pos_num = jnp.sum(n_in * n_pos, axis=0, keepdims=True)
    pos_sim = pos_num / jnp.maximum(rni * rnp, 1e-8)

    def seg_and_prob(l):
        m = jnp.max(l, axis=0, keepdims=True)
        e = jnp.exp(l - m)
        s = jnp.sum(e, axis=0, keepdims=True)
        p = e / s
        pm = jnp.max(p, axis=0, keepdims=True)
        cls_iota = _iota_f32( l.shape, 0)
        seg = jnp.min(jnp.where(p == pm, cls_iota, float(NC)), axis=0,
                      keepdims=True)
        return seg, pm

    seg_in, _ = seg_and_prob(il_ref[...])
    seg_neg, neg_prob = seg_and_prob(nl_ref[...])

    cvec = _iota_f32( (NC, N), 0)
    V0 = jnp.where(seg_neg != cvec, jnp.broadcast_to(neg_prob, (NC, N)), 0.0)

    lane = _iota_f32( (NC, N), 1)

    def body(k, carry):
        V, Rk = carry
        m = jnp.max(V, axis=1, keepdims=True)
        idx = jnp.min(jnp.where(V == m, lane, float(N)), axis=1, keepdims=True)
        hit = lane == idx
        Rk = jnp.where(hit, jnp.float32(k + 1), Rk)
        V = jnp.where(hit, -1.0, V)
        return V, Rk

    _, Rank = jax.lax.fori_loop(0, K, body, (V0, jnp.zeros((NC, N), jnp.float32)))

    row = _iota_f32( (R, N), 0)
    blk = jnp.floor(row / KPAD)
    rk = jnp.mod(row, KPAD) + 1.0
    rexp = jnp.zeros((R, N), jnp.float32)
    for c in range(NC):
        rexp = rexp + jnp.where(blk == float(c),
                                jnp.broadcast_to(Rank[c:c + 1, :], (R, N)), 0.0)
    P = (jnp.abs(rexp - rk) < 0.5).astype(jnp.float32)

    g_raw = jax.lax.dot_general(P, xneg_ref[...], (((1,), (1,)), ((), ())),
                                preferred_element_type=jnp.float32,
                                precision=jax.lax.Precision.HIGHEST)
    gn = jnp.sqrt(jnp.sum(g_raw * g_raw, axis=1, keepdims=True))
    G = g_raw / jnp.maximum(gn, 1e-12)
    rnG = jnp.sqrt(jnp.sum(G * G, axis=1, keepdims=True))

    kvalid = (jnp.mod(_iota_f32( (R, 1), 0), KPAD)
              < float(K)).astype(jnp.float32)
    blk_col = jnp.floor(_iota_f32( (R, 1), 0) / KPAD)

    def exp_sum(n_tok, rn_tok):
        S = jax.lax.dot_general(G, n_tok, (((1,), (0,)), ((), ())),
                                preferred_element_type=jnp.float32,
                                precision=jax.lax.Precision.HIGHEST)
        sim = S / jnp.maximum(rnG * rn_tok, 1e-8)
        E = jnp.exp(sim / TAU)
        msk = kvalid * (blk_col == seg_in).astype(jnp.float32)
        return jnp.sum(E * msk, axis=0, keepdims=True)

    nom = jnp.exp(pos_sim / TAU)
    den1 = exp_sum(n_in, rni) + nom
    den2 = exp_sum(n_pos, rnp) + nom
    l1 = -jnp.log(nom / (den1 + 1e-8))
    l2 = -jnp.log(nom / (den2 + 1e-8))
    out_ref[...] = jnp.sum(l1 + l2, axis=1, keepdims=True) / float(N)


def kernel(input, positive, negative, input_logits, negative_logits):
    B, C, H, W = input.shape
    N = B * H * W
    NC = input_logits.shape[1]
    xin = jnp.transpose(input, (1, 0, 2, 3)).reshape(C, N)
    xpos = jnp.transpose(positive, (1, 0, 2, 3)).reshape(C, N)
    xneg = jnp.transpose(negative, (1, 0, 2, 3)).reshape(C, N)
    il = jnp.transpose(input_logits, (1, 0, 2, 3)).reshape(NC, N)
    nl = jnp.transpose(negative_logits, (1, 0, 2, 3)).reshape(NC, N)

    out = pl.pallas_call(
        _loss_kernel,
        out_shape=jax.ShapeDtypeStruct((1, 1), jnp.float32),
    )(xin, xpos, xneg, il, nl)
    return out.reshape(())


# speedup vs baseline: 84.7397x; 1.4481x over previous
import jax
import jax.numpy as jnp
from jax.experimental import pallas as pl

TAU = 0.07
K = 50
KPAD = 64


def _iota_f32(shape, dim):
    return jax.lax.broadcasted_iota(jnp.int32, shape, dim).astype(jnp.float32)


def _loss_kernel(xin_ref, xpos_ref, xneg_ref, il_ref, nl_ref, out_ref):
    B, C, HW = xin_ref.shape
    N = B * HW
    NC = il_ref.shape[1]
    R = NC * KPAD

    xin = jnp.concatenate([xin_ref[b] for b in range(B)], axis=1)
    xpos = jnp.concatenate([xpos_ref[b] for b in range(B)], axis=1)
    il = jnp.concatenate([il_ref[b] for b in range(B)], axis=1)
    nl = jnp.concatenate([nl_ref[b] for b in range(B)], axis=1)

    rn_in = jnp.sqrt(jnp.sum(xin * xin, axis=0, keepdims=True))
    rn_pos = jnp.sqrt(jnp.sum(xpos * xpos, axis=0, keepdims=True))
    pos_num = jnp.sum(xin * xpos, axis=0, keepdims=True)
    pos_sim = pos_num / (jnp.maximum(rn_in, 1e-12) *
                         jnp.maximum(rn_pos, 1e-12))

    def seg_and_prob(l):
        m = jnp.max(l, axis=0, keepdims=True)
        e = jnp.exp(l - m)
        s = jnp.sum(e, axis=0, keepdims=True)
        p = e / s
        pm = jnp.max(p, axis=0, keepdims=True)
        cls_iota = _iota_f32(l.shape, 0)
        seg = jnp.min(jnp.where(p == pm, cls_iota, float(NC)), axis=0,
                      keepdims=True)
        return seg, pm

    seg_in, _ = seg_and_prob(il)
    seg_neg, neg_prob = seg_and_prob(nl)

    cvec = _iota_f32((NC, N), 0)
    V = jnp.where(seg_neg != cvec, jnp.broadcast_to(neg_prob, (NC, N)), 0.0)
    lane = _iota_f32((NC, N), 1)

    Rank = jnp.zeros((NC, N), jnp.float32)
    for k in range(K):
        m = jnp.max(V, axis=1, keepdims=True)
        idx = jnp.min(jnp.where(V == m, lane, float(N)), axis=1, keepdims=True)
        hit = lane == idx
        Rank = jnp.where(hit, jnp.float32(k + 1), Rank)
        V = jnp.where(hit, -1.0, V)

    row = _iota_f32((R, N), 0)
    blk = jnp.floor(row / KPAD)
    rk = jnp.mod(row, KPAD) + 1.0
    rexp = jnp.zeros((R, N), jnp.float32)
    for c in range(NC):
        rexp = rexp + jnp.where(blk == float(c),
                                jnp.broadcast_to(Rank[c:c + 1, :], (R, N)), 0.0)
    P = (jnp.abs(rexp - rk) < 0.5).astype(jnp.float32)

    xneg = jnp.concatenate([xneg_ref[b] for b in range(B)], axis=1)
    g_raw = jax.lax.dot_general(P, xneg, (((1,), (1,)), ((), ())),
                                preferred_element_type=jnp.float32)
    gn = jnp.maximum(jnp.sqrt(jnp.sum(g_raw * g_raw, axis=1, keepdims=True)),
                     1e-12)

    kvalid = jnp.mod(_iota_f32((R, 1), 0), KPAD) < float(K)
    blk_col = jnp.floor(_iota_f32((R, 1), 0) / KPAD)

    rn_in_c = jnp.maximum(rn_in, 1e-12)
    rn_pos_c = jnp.maximum(rn_pos, 1e-12)

    def exp_sum(x_tok, rn_tok):
        S = jax.lax.dot_general(g_raw, x_tok, (((1,), (0,)), ((), ())),
                                preferred_element_type=jnp.float32)
        sim = S / (gn * rn_tok)
        msk = kvalid & (blk_col == seg_in)
        E = jnp.exp(jnp.where(msk, sim / TAU, -1e30))
        return jnp.sum(E, axis=0, keepdims=True)

    nom = jnp.exp(pos_sim / TAU)
    den1 = exp_sum(xin, rn_in_c) + nom
    den2 = exp_sum(xpos, rn_pos_c) + nom
    l1 = -jnp.log(nom / (den1 + 1e-8))
    l2 = -jnp.log(nom / (den2 + 1e-8))
    out_ref[...] = jnp.sum(l1 + l2, axis=1, keepdims=True) / float(N)


def kernel(input, positive, negative, input_logits, negative_logits):
    B, C, H, W = input.shape
    NC = input_logits.shape[1]
    out = pl.pallas_call(
        _loss_kernel,
        out_shape=jax.ShapeDtypeStruct((1, 1), jnp.float32),
    )(input.reshape(B, C, H * W), positive.reshape(B, C, H * W),
      negative.reshape(B, C, H * W), input_logits.reshape(B, NC, H * W),
      negative_logits.reshape(B, NC, H * W))
    return out.reshape(())


# speedup vs baseline: 93.3465x; 1.1016x over previous
import jax
import jax.numpy as jnp
from jax.experimental import pallas as pl

TAU = 0.07
K = 50
KPAD = 64


def _iota_f32(shape, dim):
    return jax.lax.broadcasted_iota(jnp.int32, shape, dim).astype(jnp.float32)


def _rowsum_mxu(x):
    ones = jnp.ones((1, x.shape[0]), jnp.float32)
    return jax.lax.dot_general(ones, x, (((1,), (0,)), ((), ())),
                               preferred_element_type=jnp.float32)


def _loss_kernel(xin_ref, xpos_ref, xneg_ref, il_ref, nl_ref, out_ref):
    B, C, HW = xin_ref.shape
    N = B * HW
    NC = il_ref.shape[1]
    R = NC * KPAD
    LN = 128
    SB = N // LN

    il = jnp.concatenate([il_ref[b] for b in range(B)], axis=1)
    nl = jnp.concatenate([nl_ref[b] for b in range(B)], axis=1)

    def seg_and_prob(l):
        m = jnp.max(l, axis=0, keepdims=True)
        e = jnp.exp(l - m)
        s = jnp.sum(e, axis=0, keepdims=True)
        p = e / s
        pm = jnp.max(p, axis=0, keepdims=True)
        cls_iota = _iota_f32(l.shape, 0)
        seg = jnp.min(jnp.where(p == pm, cls_iota, float(NC)), axis=0,
                      keepdims=True)
        return seg, pm

    seg_in, _ = seg_and_prob(il)
    seg_neg, neg_prob = seg_and_prob(nl)

    cvec = _iota_f32((NC, N), 0)
    V = jnp.where(seg_neg != cvec, jnp.broadcast_to(neg_prob, (NC, N)),
                  0.0).reshape(NC, SB, LN)
    lin = (_iota_f32((NC, SB, LN), 1) * LN +
           _iota_f32((NC, SB, LN), 2))

    Rank = jnp.zeros((NC, SB, LN), jnp.float32)
    for k in range(K):
        m = jnp.max(V, axis=(1, 2), keepdims=True)
        idx = jnp.min(jnp.where(V == m, lin, float(N)), axis=(1, 2),
                      keepdims=True)
        hit = lin == idx
        Rank = jnp.where(hit, jnp.float32(k + 1), Rank)
        V = jnp.where(hit, -1.0, V)
    Rank = Rank.reshape(NC, N)

    row = _iota_f32((R, N), 0)
    blk = jnp.floor(row / KPAD)
    rk = jnp.mod(row, KPAD) + 1.0
    rexp = jnp.zeros((R, N), jnp.float32)
    for c in range(NC):
        rexp = rexp + jnp.where(blk == float(c),
                                jnp.broadcast_to(Rank[c:c + 1, :], (R, N)), 0.0)
    P = (jnp.abs(rexp - rk) < 0.5).astype(jnp.float32)

    g_raw = jnp.zeros((R, C), jnp.float32)
    for b in range(B):
        g_raw = g_raw + jax.lax.dot_general(
            P[:, b * HW:(b + 1) * HW], xneg_ref[b], (((1,), (1,)), ((), ())),
            preferred_element_type=jnp.float32)
    gn = jnp.maximum(jnp.sqrt(jnp.sum(g_raw * g_raw, axis=1, keepdims=True)),
                     1e-12)
    g_scaled = g_raw / gn

    kvalid = jnp.mod(_iota_f32((KPAD, 1), 0), KPAD) < float(K)

    total = jnp.zeros((1, 1), jnp.float32)
    for b in range(B):
        xb = xin_ref[b]
        pb = xpos_ref[b]
        seg_b = seg_in[:, b * HW:(b + 1) * HW]
        rn2_in = _rowsum_mxu(xb * xb)
        rn2_pos = _rowsum_mxu(pb * pb)
        cross = _rowsum_mxu(xb * pb)
        rin = jnp.maximum(jnp.sqrt(rn2_in), 1e-12)
        rpos = jnp.maximum(jnp.sqrt(rn2_pos), 1e-12)
        pos_sim = cross / (rin * rpos)
        nom = jnp.exp(pos_sim / TAU)

        def den_of(x_tok, rn_tok):
            S = jax.lax.dot_general(g_scaled, x_tok, (((1,), (0,)), ((), ())),
                                    preferred_element_type=jnp.float32)
            sel = S[0:KPAD]
            for c in range(1, NC):
                sel = jnp.where(seg_b == float(c),
                                S[c * KPAD:(c + 1) * KPAD], sel)
            E = jnp.where(kvalid, jnp.exp(sel / (rn_tok * TAU)), 0.0)
            return jnp.sum(E, axis=0, keepdims=True)

        den1 = den_of(xb, rin) + nom
        den2 = den_of(pb, rpos) + nom
        l12 = -jnp.log(nom / (den1 + 1e-8)) - jnp.log(nom / (den2 + 1e-8))
        total = total + jnp.sum(l12, axis=1, keepdims=True)

    out_ref[...] = total / float(N)


def kernel(input, positive, negative, input_logits, negative_logits):
    B, C, H, W = input.shape
    NC = input_logits.shape[1]
    out = pl.pallas_call(
        _loss_kernel,
        out_shape=jax.ShapeDtypeStruct((1, 1), jnp.float32),
    )(input.reshape(B, C, H * W), positive.reshape(B, C, H * W),
      negative.reshape(B, C, H * W), input_logits.reshape(B, NC, H * W),
      negative_logits.reshape(B, NC, H * W))
    return out.reshape(())


# speedup vs baseline: 118.1655x; 1.2659x over previous
import jax
import jax.numpy as jnp
from jax.experimental import pallas as pl
from jax.experimental.pallas import tpu as pltpu

TAU = 0.07
K = 50
KPAD = 64
LN = 128
ONE_F32_BITS_PLUS = 0x3F800001


def _iota_f32(shape, dim):
    return jax.lax.broadcasted_iota(jnp.int32, shape, dim).astype(jnp.float32)


def _rowsum_mxu(x):
    ones = jnp.ones((1, x.shape[0]), jnp.float32)
    return jax.lax.dot_general(ones, x, (((1,), (0,)), ((), ())),
                               preferred_element_type=jnp.float32)


def _loss_kernel(xin_h, xpos_h, xneg_h, il_h, nl_h, out_ref,
                 xin_v, xpos_v, xneg_v, il_v, nl_v,
                 s_il, s_nl, s_neg, s_in, s_pos):
    B, C, HW = xin_v.shape
    N = B * HW
    NC = il_v.shape[1]
    R = NC * KPAD
    SB = N // LN

    cp_il = pltpu.make_async_copy(il_h, il_v, s_il)
    cp_nl = pltpu.make_async_copy(nl_h, nl_v, s_nl)
    cp_neg = pltpu.make_async_copy(xneg_h, xneg_v, s_neg)
    cp_in = pltpu.make_async_copy(xin_h, xin_v, s_in)
    cp_pos = pltpu.make_async_copy(xpos_h, xpos_v, s_pos)
    cp_il.start()
    cp_nl.start()
    cp_neg.start()
    cp_in.start()
    cp_pos.start()

    cp_il.wait()
    cp_nl.wait()
    il = jnp.concatenate([il_v[b] for b in range(B)], axis=1)
    nl = jnp.concatenate([nl_v[b] for b in range(B)], axis=1)

    def seg_and_prob(l):
        m = jnp.max(l, axis=0, keepdims=True)
        e = jnp.exp(l - m)
        s = jnp.sum(e, axis=0, keepdims=True)
        p = e / s
        pm = jnp.max(p, axis=0, keepdims=True)
        cls_iota = _iota_f32(l.shape, 0)
        seg = jnp.min(jnp.where(p == pm, cls_iota, float(NC)), axis=0,
                      keepdims=True)
        return seg, pm

    seg_in, _ = seg_and_prob(il)
    seg_neg, neg_prob = seg_and_prob(nl)

    cvec = _iota_f32((NC, N), 0)
    V = jnp.where(seg_neg != cvec, jnp.broadcast_to(neg_prob, (NC, N)),
                  0.0).reshape(NC, SB, LN)
    Vb = jax.lax.bitcast_convert_type(V, jnp.int32)

    lo = jnp.full((NC, 1, 1), -1, jnp.int32)
    hi = jnp.full((NC, 1, 1), ONE_F32_BITS_PLUS, jnp.int32)
    for _ in range(30):
        mid = lo + ((hi - lo) >> 1)
        cnt = jnp.sum((Vb > mid).astype(jnp.float32), axis=(1, 2),
                      keepdims=True)
        pred = cnt >= float(K)
        lo = jnp.where(pred, mid, lo)
        hi = jnp.where(pred, hi, mid)
    bstar = hi

    gt = Vb > bstar
    tie = Vb == bstar
    m_gt = jnp.sum(gt.astype(jnp.float32), axis=(1, 2), keepdims=True)
    fill = float(K) - m_gt

    U = (jax.lax.broadcasted_iota(jnp.int32, (LN, LN), 0) <=
         jax.lax.broadcasted_iota(jnp.int32, (LN, LN), 1)).astype(jnp.float32)
    r0 = jax.lax.broadcasted_iota(jnp.int32, (NC * SB, NC * SB), 0)
    r1 = jax.lax.broadcasted_iota(jnp.int32, (NC * SB, NC * SB), 1)
    Tm = ((r1 // SB == r0 // SB) & (r1 < r0)).astype(jnp.float32)

    def prefix_incl(mask_f):
        m2 = mask_f.reshape(NC * SB, LN)
        pref = jax.lax.dot_general(m2, U, (((1,), (0,)), ((), ())),
                                   preferred_element_type=jnp.float32)
        rt = pref[:, LN - 1:LN]
        off = jax.lax.dot_general(Tm, rt, (((1,), (0,)), ((), ())),
                                  preferred_element_type=jnp.float32)
        return (pref + off).reshape(NC, SB, LN)

    tie_f = tie.astype(jnp.float32)
    tie_excl = prefix_incl(tie_f) - tie_f
    sel = gt | (tie & (tie_excl < fill))
    sel_f = sel.astype(jnp.float32)
    rank = jnp.where(sel, prefix_incl(sel_f), 0.0)

    rk64 = _iota_f32((KPAD, 1), 0) + 1.0
    cp_neg.wait()
    g_blocks = []
    for c in range(NC):
        rank_row = rank[c].reshape(1, N)
        Pc = (rank_row == rk64).astype(jnp.float32)
        g_c = jnp.zeros((KPAD, C), jnp.float32)
        for b in range(B):
            g_c = g_c + jax.lax.dot_general(
                Pc[:, b * HW:(b + 1) * HW], xneg_v[b], (((1,), (1,)), ((), ())),
                preferred_element_type=jnp.float32)
        g_blocks.append(g_c)
    g_raw = jnp.concatenate(g_blocks, axis=0)
    gn = jnp.maximum(jnp.sqrt(jnp.sum(g_raw * g_raw, axis=1, keepdims=True)),
                     1e-12)
    g_scaled = g_raw / gn

    kvalid = jnp.mod(_iota_f32((KPAD, 1), 0), KPAD) < float(K)

    cp_in.wait()
    cp_pos.wait()
    total = jnp.zeros((1, 1), jnp.float32)
    for b in range(B):
        xb = xin_v[b]
        pb = xpos_v[b]
        seg_b = seg_in[:, b * HW:(b + 1) * HW]
        rin = jnp.maximum(jnp.sqrt(_rowsum_mxu(xb * xb)), 1e-12)
        rpos = jnp.maximum(jnp.sqrt(_rowsum_mxu(pb * pb)), 1e-12)
        pos_sim = _rowsum_mxu(xb * pb) / (rin * rpos)
        nom = jnp.exp(pos_sim / TAU)

        def den_of(x_tok, rn_tok):
            S = jax.lax.dot_general(g_scaled, x_tok, (((1,), (0,)), ((), ())),
                                    preferred_element_type=jnp.float32)
            sel_S = S[0:KPAD]
            for c in range(1, NC):
                sel_S = jnp.where(seg_b == float(c),
                                  S[c * KPAD:(c + 1) * KPAD], sel_S)
            E = jnp.where(kvalid, jnp.exp(sel_S / (rn_tok * TAU)), 0.0)
            return jnp.sum(E, axis=0, keepdims=True)

        den1 = den_of(xb, rin) + nom
        den2 = den_of(pb, rpos) + nom
        l12 = -jnp.log(nom / (den1 + 1e-8)) - jnp.log(nom / (den2 + 1e-8))
        total = total + jnp.sum(l12, axis=1, keepdims=True)

    out_ref[...] = total / float(N)


def kernel(input, positive, negative, input_logits, negative_logits):
    B, C, H, W = input.shape
    HW = H * W
    NC = input_logits.shape[1]
    out = pl.pallas_call(
        _loss_kernel,
        out_shape=jax.ShapeDtypeStruct((1, 1), jnp.float32),
        in_specs=[pl.BlockSpec(memory_space=pl.ANY)] * 5,
        out_specs=pl.BlockSpec(memory_space=pltpu.MemorySpace.VMEM),
        scratch_shapes=[pltpu.VMEM((B, C, HW), jnp.float32)] * 3 +
                       [pltpu.VMEM((B, NC, HW), jnp.float32)] * 2 +
                       [pltpu.SemaphoreType.DMA] * 5,
    )(input.reshape(B, C, HW), positive.reshape(B, C, HW),
      negative.reshape(B, C, HW), input_logits.reshape(B, NC, HW),
      negative_logits.reshape(B, NC, HW))
    return out.reshape(())
